# Initial kernel scaffold; baseline (speedup 1.0000x reference)
import functools

import jax
import jax.numpy as jnp
from jax import lax
from jax.experimental import pallas as pl
from jax.experimental.pallas import tpu as pltpu
from jax.experimental.pallas import tpu_sc as plsc

N_NODES = 10000
N_EDGES = 320000
D = 128
H = 8
HD = D // H

NC = 2
NS = 16
NW = NC * NS
EPT = N_EDGES // NW
C = 80
NCHUNK = EPT // C
G = C // 16


def _qkv_body(x_ref, wq_ref, wk_ref, wv_ref, bq_ref, bk_ref, bv_ref,
              q_ref, k_ref, v_ref):
    x = x_ref[...]
    q_ref[...] = jnp.dot(x, wq_ref[...], preferred_element_type=jnp.float32) + bq_ref[...]
    k_ref[...] = jnp.dot(x, wk_ref[...], preferred_element_type=jnp.float32) + bk_ref[...]
    v_ref[...] = jnp.dot(x, wv_ref[...], preferred_element_type=jnp.float32) + bv_ref[...]


def _qkv_project(x, wqt, wkt, wvt, bq, bk, bv):
    B = 2000
    grid = (N_NODES // B,)
    row_spec = pl.BlockSpec((B, D), lambda i: (i, 0))
    w_spec = pl.BlockSpec((D, D), lambda i: (0, 0))
    b_spec = pl.BlockSpec((1, D), lambda i: (0, 0))
    return pl.pallas_call(
        _qkv_body,
        grid=grid,
        in_specs=[row_spec, w_spec, w_spec, w_spec, b_spec, b_spec, b_spec],
        out_specs=[row_spec, row_spec, row_spec],
        out_shape=[jax.ShapeDtypeStruct((N_NODES, D), jnp.float32)] * 3,
    )(x, wqt, wkt, wvt, bq, bk, bv)


def _edge_body(edges_hbm, q_hbm, k_hbm, v_hbm, zn_hbm, zd_hbm,
               numer_out, denom_out,
               src_v, tgt_v, q_rows, k_rows, v_rows, ex_v,
               numer_sh, denom_sh, sem):
    cid = lax.axis_index("c")
    sid = lax.axis_index("s")
    rpt = N_NODES // NS

    pltpu.sync_copy(zn_hbm.at[pl.ds(sid * rpt, rpt)],
                    numer_sh.at[pl.ds(sid * rpt, rpt)])
    pltpu.sync_copy(zd_hbm.at[pl.ds(sid * rpt, rpt)],
                    denom_sh.at[pl.ds(sid * rpt, rpt)])
    plsc.subcore_barrier()

    wid = sid * NC + cid
    base = wid * EPT

    def chunk_body(i, carry):
        off = base + i * C
        pltpu.sync_copy(edges_hbm.at[0, pl.ds(off, C)], src_v)
        pltpu.sync_copy(edges_hbm.at[1, pl.ds(off, C)], tgt_v)
        cq = pltpu.async_copy(q_hbm.at[tgt_v], q_rows, sem)
        ck = pltpu.async_copy(k_hbm.at[src_v], k_rows, sem)
        cv = pltpu.async_copy(v_hbm.at[src_v], v_rows, sem)
        cq.wait()
        ck.wait()
        cv.wait()

        def group_body(g, carry2):
            eidx = lax.iota(jnp.int32, 16) + g * 16
            for h in range(H):
                acc = jnp.zeros((16,), jnp.float32)
                for dd in range(HD):
                    col = jnp.full((16,), h * HD + dd, jnp.int32)
                    qv = plsc.load_gather(q_rows, [eidx, col])
                    kv = plsc.load_gather(k_rows, [eidx, col])
                    acc = acc + qv * kv
                ex = jnp.exp(acc)
                plsc.store_scatter(ex_v, [eidx, jnp.full((16,), h, jnp.int32)], ex)
                for dd in range(HD):
                    col = jnp.full((16,), h * HD + dd, jnp.int32)
                    vv = plsc.load_gather(v_rows, [eidx, col])
                    plsc.store_scatter(v_rows, [eidx, col], vv * ex)
            return carry2

        lax.fori_loop(0, G, group_body, 0)

        pltpu.sync_copy(v_rows, numer_sh.at[tgt_v], add=True)
        pltpu.sync_copy(ex_v, denom_sh.at[tgt_v], add=True)
        return carry

    lax.fori_loop(0, NCHUNK, chunk_body, 0)
    plsc.subcore_barrier()

    pltpu.sync_copy(numer_sh.at[pl.ds(sid * rpt, rpt)],
                    numer_out.at[cid, pl.ds(sid * rpt, rpt)])
    pltpu.sync_copy(denom_sh.at[pl.ds(sid * rpt, rpt)],
                    denom_out.at[cid, pl.ds(sid * rpt, rpt)])


@functools.partial(
    pl.kernel,
    mesh=plsc.VectorSubcoreMesh(core_axis_name="c", subcore_axis_name="s"),
    out_type=[jax.ShapeDtypeStruct((NC, N_NODES, D), jnp.float32),
              jax.ShapeDtypeStruct((NC, N_NODES, H), jnp.float32)],
    scratch_types=[
        pltpu.VMEM((C,), jnp.int32),
        pltpu.VMEM((C,), jnp.int32),
        pltpu.VMEM((C, D), jnp.float32),
        pltpu.VMEM((C, D), jnp.float32),
        pltpu.VMEM((C, D), jnp.float32),
        pltpu.VMEM((C, H), jnp.float32),
        pltpu.VMEM_SHARED((N_NODES, D), jnp.float32),
        pltpu.VMEM_SHARED((N_NODES, H), jnp.float32),
        pltpu.SemaphoreType.DMA,
    ],
)
def _edge_kernel(edges_hbm, q_hbm, k_hbm, v_hbm, zn_hbm, zd_hbm,
                 numer_out, denom_out, *scratch):
    _edge_body(edges_hbm, q_hbm, k_hbm, v_hbm, zn_hbm, zd_hbm,
               numer_out, denom_out, *scratch)


def _out_body(numer_ref, denom_ref, e_ref, w_ref, b_ref, o_ref):
    nu = numer_ref[...]
    de = denom_ref[...]
    nsum = nu[0] + nu[1]
    dsum = de[0] + de[1]
    dexp = jnp.dot(dsum, e_ref[...], preferred_element_type=jnp.float32)
    attn = jnp.where(dexp > 0.0, nsum / dexp, 0.0)
    o_ref[...] = jnp.dot(attn, w_ref[...], preferred_element_type=jnp.float32) + b_ref[...]


def _out_project(numer, denom, e_mat, wot, b_out2d):
    B = 2000
    grid = (N_NODES // B,)
    return pl.pallas_call(
        _out_body,
        grid=grid,
        in_specs=[pl.BlockSpec((NC, B, D), lambda i: (0, i, 0)),
                  pl.BlockSpec((NC, B, H), lambda i: (0, i, 0)),
                  pl.BlockSpec((H, D), lambda i: (0, 0)),
                  pl.BlockSpec((D, D), lambda i: (0, 0)),
                  pl.BlockSpec((1, D), lambda i: (0, 0))],
        out_specs=pl.BlockSpec((B, D), lambda i: (i, 0)),
        out_shape=jax.ShapeDtypeStruct((N_NODES, D), jnp.float32),
    )(numer, denom, e_mat, wot, b_out2d)


def kernel(node_states, edges, W_qkv, b_qkv, W_out, b_out):
    scale = float(HD) ** -0.5
    wqt = W_qkv[0:D].T * scale
    wkt = W_qkv[D:2 * D].T
    wvt = W_qkv[2 * D:3 * D].T
    bq = (b_qkv[0:D] * scale).reshape(1, D)
    bk = b_qkv[D:2 * D].reshape(1, D)
    bv = b_qkv[2 * D:3 * D].reshape(1, D)

    q, k, v = _qkv_project(node_states, wqt, wkt, wvt, bq, bk, bv)

    zn = jnp.zeros((N_NODES, D), jnp.float32)
    zd = jnp.zeros((N_NODES, H), jnp.float32)
    numer, denom = _edge_kernel(edges, q, k, v, zn, zd)

    e_mat = jnp.repeat(jnp.eye(H, dtype=jnp.float32), HD, axis=1)
    out = _out_project(numer, denom, e_mat, W_out.T, b_out.reshape(1, D))
    return out


# speedup vs baseline: 2.0028x; 2.0028x over previous
import functools

import jax
import jax.numpy as jnp
from jax import lax
from jax.experimental import pallas as pl
from jax.experimental.pallas import tpu as pltpu
from jax.experimental.pallas import tpu_sc as plsc

N_NODES = 10000
N_EDGES = 320000
D = 128
H = 8
HD = D // H

NC = 2
NS = 16
NW = NC * NS
EPT = N_EDGES // NW
C = 80
NCHUNK = EPT // C
G = C // 16
NP = 10240


def _qkv_body(x_ref, wq_ref, wk_ref, wv_ref, bq_ref, bk_ref, bv_ref,
              q_ref, k_ref, v_ref):
    x = x_ref[...]
    q_ref[...] = jnp.dot(x, wq_ref[...], preferred_element_type=jnp.float32) + bq_ref[...]
    k_ref[...] = jnp.dot(x, wk_ref[...], preferred_element_type=jnp.float32) + bk_ref[...]
    v_ref[...] = jnp.dot(x, wv_ref[...], preferred_element_type=jnp.float32) + bv_ref[...]


def _qkv_project(x, wqt, wkt, wvt, bq, bk, bv):
    B = 2000
    grid = (N_NODES // B,)
    row_spec = pl.BlockSpec((B, D), lambda i: (i, 0))
    w_spec = pl.BlockSpec((D, D), lambda i: (0, 0))
    b_spec = pl.BlockSpec((1, D), lambda i: (0, 0))
    return pl.pallas_call(
        _qkv_body,
        grid=grid,
        in_specs=[row_spec, w_spec, w_spec, w_spec, b_spec, b_spec, b_spec],
        out_specs=[row_spec, row_spec, row_spec],
        out_shape=[jax.ShapeDtypeStruct((N_NODES, D), jnp.float32)] * 3,
    )(x, wqt, wkt, wvt, bq, bk, bv)


def _edge_body(src_hbm, tgt_hbm, q_hbm, k_hbm, v_hbm, zn_hbm, zd_hbm,
               numer_out, denom_out,
               src_v, tgt_v, q_rows, k_rows, v_rows, ex_v,
               numer_sh, denom_sh, sem):
    cid = lax.axis_index("c")
    sid = lax.axis_index("s")
    rpt = NP // NS

    pltpu.sync_copy(zn_hbm.at[pl.ds(sid * rpt, rpt)],
                    numer_sh.at[pl.ds(sid * rpt, rpt)])
    pltpu.sync_copy(zd_hbm.at[pl.ds(sid * rpt, rpt)],
                    denom_sh.at[pl.ds(sid * rpt, rpt)])
    plsc.subcore_barrier()

    wid = sid * NC + cid
    base = wid * EPT

    def chunk_body(i, carry):
        off = base + i * C
        pltpu.sync_copy(src_hbm.at[pl.ds(off, C)], src_v)
        pltpu.sync_copy(tgt_hbm.at[pl.ds(off, C)], tgt_v)
        cq = pltpu.async_copy(q_hbm.at[tgt_v], q_rows, sem)
        ck = pltpu.async_copy(k_hbm.at[src_v], k_rows, sem)
        cv = pltpu.async_copy(v_hbm.at[src_v], v_rows, sem)
        cq.wait()
        ck.wait()
        cv.wait()

        def group_body(g, carry2):
            eidx = lax.iota(jnp.int32, 16) + g * 16
            for h in range(H):
                acc = jnp.zeros((16,), jnp.float32)
                for dd in range(HD):
                    col = jnp.full((16,), h * HD + dd, jnp.int32)
                    qv = plsc.load_gather(q_rows, [eidx, col])
                    kv = plsc.load_gather(k_rows, [eidx, col])
                    acc = acc + qv * kv
                ex = jnp.exp(acc)
                plsc.store_scatter(ex_v, [eidx, jnp.full((16,), h, jnp.int32)], ex)
                for dd in range(HD):
                    col = jnp.full((16,), h * HD + dd, jnp.int32)
                    vv = plsc.load_gather(v_rows, [eidx, col])
                    plsc.store_scatter(v_rows, [eidx, col], vv * ex)
            return carry2

        lax.fori_loop(0, G, group_body, 0)

        pltpu.sync_copy(v_rows, numer_sh.at[tgt_v], add=True)
        pltpu.sync_copy(ex_v, denom_sh.at[tgt_v], add=True)
        return carry

    lax.fori_loop(0, NCHUNK, chunk_body, 0)
    plsc.subcore_barrier()

    pltpu.sync_copy(numer_sh.at[pl.ds(sid * rpt, rpt)],
                    numer_out.at[cid, pl.ds(sid * rpt, rpt)])
    pltpu.sync_copy(denom_sh.at[pl.ds(sid * rpt, rpt)],
                    denom_out.at[cid, pl.ds(sid * rpt, rpt)])


@functools.cache
def _make_edge_kernel():
    @functools.partial(
        pl.kernel,
        mesh=plsc.VectorSubcoreMesh(core_axis_name="c", subcore_axis_name="s"),
        out_type=[jax.ShapeDtypeStruct((NC, NP, D), jnp.float32),
                  jax.ShapeDtypeStruct((NC, NP, H), jnp.float32)],
        scratch_types=[
            pltpu.VMEM((C,), jnp.int32),
            pltpu.VMEM((C,), jnp.int32),
            pltpu.VMEM((C, D), jnp.float32),
            pltpu.VMEM((C, D), jnp.float32),
            pltpu.VMEM((C, D), jnp.float32),
            pltpu.VMEM((C, H), jnp.float32),
            pltpu.VMEM_SHARED((NP, D), jnp.float32),
            pltpu.VMEM_SHARED((NP, H), jnp.float32),
            pltpu.SemaphoreType.DMA,
        ],
        compiler_params=pltpu.CompilerParams(
            needs_layout_passes=False, use_tc_tiling_on_sc=False),
    )
    def _edge_kernel(src_hbm, tgt_hbm, q_hbm, k_hbm, v_hbm, zn_hbm, zd_hbm,
                     numer_out, denom_out, *scratch):
        _edge_body(src_hbm, tgt_hbm, q_hbm, k_hbm, v_hbm, zn_hbm, zd_hbm,
                   numer_out, denom_out, *scratch)

    return _edge_kernel


def _out_body(numer_ref, denom_ref, e_ref, w_ref, b_ref, o_ref):
    nu = numer_ref[...]
    de = denom_ref[...]
    nsum = nu[0] + nu[1]
    dsum = de[0] + de[1]
    dexp = jnp.dot(dsum, e_ref[...], preferred_element_type=jnp.float32)
    attn = jnp.where(dexp > 0.0, nsum / dexp, 0.0)
    o_ref[...] = jnp.dot(attn, w_ref[...], preferred_element_type=jnp.float32) + b_ref[...]


def _out_project(numer, denom, e_mat, wot, b_out2d):
    B = 2000
    grid = (N_NODES // B,)
    return pl.pallas_call(
        _out_body,
        grid=grid,
        in_specs=[pl.BlockSpec((NC, B, D), lambda i: (0, i, 0)),
                  pl.BlockSpec((NC, B, H), lambda i: (0, i, 0)),
                  pl.BlockSpec((H, D), lambda i: (0, 0)),
                  pl.BlockSpec((D, D), lambda i: (0, 0)),
                  pl.BlockSpec((1, D), lambda i: (0, 0))],
        out_specs=pl.BlockSpec((B, D), lambda i: (i, 0)),
        out_shape=jax.ShapeDtypeStruct((N_NODES, D), jnp.float32),
    )(numer, denom, e_mat, wot, b_out2d)


def kernel(node_states, edges, W_qkv, b_qkv, W_out, b_out):
    scale = float(HD) ** -0.5
    wqt = W_qkv[0:D].T * scale
    wkt = W_qkv[D:2 * D].T
    wvt = W_qkv[2 * D:3 * D].T
    bq = (b_qkv[0:D] * scale).reshape(1, D)
    bk = b_qkv[D:2 * D].reshape(1, D)
    bv = b_qkv[2 * D:3 * D].reshape(1, D)

    q, k, v = _qkv_project(node_states, wqt, wkt, wvt, bq, bk, bv)

    zn = jnp.zeros((NP, D), jnp.float32)
    zd = jnp.zeros((NP, H), jnp.float32)
    numer, denom = _make_edge_kernel()(edges[0], edges[1], q, k, v, zn, zd)

    e_mat = jnp.repeat(jnp.eye(H, dtype=jnp.float32), HD, axis=1)
    out = _out_project(numer, denom, e_mat, W_out.T, b_out.reshape(1, D))
    return out
